# baseline (speedup 1.0000x reference)
import functools

import jax
import jax.numpy as jnp
from jax import lax
from jax.experimental import pallas as pl
from jax.experimental.pallas import tpu as pltpu
from jax.experimental.pallas import tpu_sc as plsc

NC = 2
NS = 16
NW = NC * NS
L = 16

B, N, D = 4, 8192, 1024
SC_ROWS = 1024
SC_START = N - SC_ROWS
TC_ROWS = SC_START
PER_W = SC_ROWS // NW
CHUNK_ROWS = 16
CHUNK = CHUNK_ROWS * D
NCH = PER_W // CHUNK_ROWS

TC_BLK = 1024


def _sc_body(x_hbm, pos_hbm, out_hbm,
             xb0, xb1, xb2, xb3, pb0, pb1,
             sx0, sx1, sx2, sx3, so0, so1, so2, so3, sp0, sp1):
    c = lax.axis_index("c")
    s = lax.axis_index("s")
    wid = s * NC + c
    pbase = (SC_START + wid * PER_W) * D

    xbs = (xb0, xb1, xb2, xb3)
    sxs = (sx0, sx1, sx2, sx3)
    sos = (so0, so1, so2, so3)
    pbs = (pb0, pb1)
    sps = (sp0, sp1)

    def poff(ci):
        return pbase + ci * CHUNK

    def xoff(ci, b):
        return b * (N * D) + poff(ci)

    def ooff(ci, b):
        return b * (SC_ROWS * D) + (wid * PER_W) * D + ci * CHUNK

    pltpu.async_copy(pos_hbm.at[pl.ds(poff(0), CHUNK)], pb0, sp0)
    for b in range(B):
        pltpu.async_copy(x_hbm.at[pl.ds(xoff(0, b), CHUNK)], xbs[b], sxs[b])

    def pair_body(cp, carry):
        for cc in range(2):
            ci = 2 * cp + cc
            if cc == 0:
                pltpu.async_copy(
                    pos_hbm.at[pl.ds(poff(ci + 1), CHUNK)], pbs[1], sps[1])
            else:
                @pl.when(cp + 1 < NCH // 2)
                def _():
                    pltpu.async_copy(
                        pos_hbm.at[pl.ds(poff(ci + 1), CHUNK)], pbs[0], sps[0])

            @pl.when(ci > 0)
            def _():
                for b in range(B):
                    pltpu.make_async_copy(
                        xbs[b], out_hbm.at[pl.ds(0, CHUNK)], sos[b]).wait()
                    pltpu.async_copy(
                        x_hbm.at[pl.ds(xoff(ci, b), CHUNK)], xbs[b], sxs[b])

            pltpu.make_async_copy(
                pos_hbm.at[pl.ds(0, CHUNK)], pbs[cc], sps[cc]).wait()

            for b in range(B):
                pltpu.make_async_copy(
                    x_hbm.at[pl.ds(0, CHUNK)], xbs[b], sxs[b]).wait()

                xb, pb = xbs[b], pbs[cc]

                @plsc.parallel_loop(0, CHUNK, step=L, unroll=8)
                def _(i):
                    sl = pl.ds(pl.multiple_of(i, L), L)
                    xb[sl] = xb[sl] + pb[sl]

                pltpu.async_copy(
                    xb, out_hbm.at[pl.ds(ooff(ci, b), CHUNK)], sos[b])
        return carry

    lax.fori_loop(0, NCH // 2, pair_body, 0)

    for b in range(B):
        pltpu.make_async_copy(
            xbs[b], out_hbm.at[pl.ds(0, CHUNK)], sos[b]).wait()


_sc_call = functools.partial(
    pl.kernel,
    out_type=jax.ShapeDtypeStruct((B * SC_ROWS * D,), jnp.float32),
    mesh=plsc.VectorSubcoreMesh(
        core_axis_name="c", subcore_axis_name="s",
        num_cores=NC, num_subcores=NS),
    scratch_types=(
        [pltpu.VMEM((CHUNK,), jnp.float32)] * 6
        + [pltpu.SemaphoreType.DMA] * 10
    ),
)(_sc_body)


def _tc_add_kernel(x_ref, pos_ref, out_ref):
    out_ref[0] = x_ref[0] + pos_ref[...]


def _tc_call(x_tc, pos_tc):
    b, n, d = x_tc.shape
    grid = (n // TC_BLK, b)
    return pl.pallas_call(
        _tc_add_kernel,
        grid=grid,
        in_specs=[
            pl.BlockSpec((1, TC_BLK, d), lambda s, bb: (bb, s, 0)),
            pl.BlockSpec((TC_BLK, d), lambda s, bb: (s, 0)),
        ],
        out_specs=pl.BlockSpec((1, TC_BLK, d), lambda s, bb: (bb, s, 0)),
        out_shape=jax.ShapeDtypeStruct((b, n, d), x_tc.dtype),
    )(x_tc, pos_tc)


def kernel(x, pos_emb):
    b, n, d = x.shape
    sc_flat = _sc_call(x.reshape(-1), pos_emb[:n].reshape(-1))
    tc_out = _tc_call(x[:, :TC_ROWS], pos_emb[:TC_ROWS])
    sc_flat, tc_out = lax.optimization_barrier((sc_flat, tc_out))
    return jnp.concatenate(
        [tc_out, sc_flat.reshape(b, SC_ROWS, d)], axis=1)


# speedup vs baseline: 1.1832x; 1.1832x over previous
import functools

import jax
import jax.numpy as jnp
from jax import lax
from jax.experimental import pallas as pl
from jax.experimental.pallas import tpu as pltpu
from jax.experimental.pallas import tpu_sc as plsc

NC = 2
NS = 16
NW = NC * NS
L = 16

B, N, D = 4, 8192, 1024
SC_BATCH = B - 1
PER_W = N // NW
CHUNK_ROWS = 8
CHUNK = CHUNK_ROWS * D
NCH = PER_W // CHUNK_ROWS
RING = 4

TC_BLK = 2048


def _sc_body(x_hbm, pos_hbm, out_hbm,
             xb0, xb1, xb2, xb3, pb0, pb1, pb2, pb3,
             sx0, sx1, sx2, sx3, so0, so1, so2, so3, sp0, sp1, sp2, sp3):
    c = lax.axis_index("c")
    s = lax.axis_index("s")
    wid = s * NC + c
    pbase = wid * (PER_W * D)

    xbs = (xb0, xb1, xb2, xb3)
    sxs = (sx0, sx1, sx2, sx3)
    sos = (so0, so1, so2, so3)
    pbs = (pb0, pb1, pb2, pb3)
    sps = (sp0, sp1, sp2, sp3)

    def poff(ci):
        return pbase + ci * CHUNK

    def xoff(ci):
        return SC_BATCH * (N * D) + poff(ci)

    for k in range(RING):
        pltpu.async_copy(pos_hbm.at[pl.ds(poff(k), CHUNK)], pbs[k], sps[k])
        pltpu.async_copy(x_hbm.at[pl.ds(xoff(k), CHUNK)], xbs[k], sxs[k])

    def group_body(g, carry):
        for k in range(RING):
            ci = RING * g + k

            @pl.when(g > 0)
            def _():
                pltpu.make_async_copy(
                    xbs[k], out_hbm.at[pl.ds(0, CHUNK)], sos[k]).wait()
                pltpu.async_copy(
                    x_hbm.at[pl.ds(xoff(ci), CHUNK)], xbs[k], sxs[k])
                pltpu.async_copy(
                    pos_hbm.at[pl.ds(poff(ci), CHUNK)], pbs[k], sps[k])

            pltpu.make_async_copy(
                pos_hbm.at[pl.ds(0, CHUNK)], pbs[k], sps[k]).wait()
            pltpu.make_async_copy(
                x_hbm.at[pl.ds(0, CHUNK)], xbs[k], sxs[k]).wait()

            xb, pb = xbs[k], pbs[k]

            @plsc.parallel_loop(0, CHUNK, step=L, unroll=8)
            def _(i):
                sl = pl.ds(pl.multiple_of(i, L), L)
                xb[sl] = xb[sl] + pb[sl]

            pltpu.async_copy(xb, out_hbm.at[pl.ds(poff(ci), CHUNK)], sos[k])
        return carry

    lax.fori_loop(0, NCH // RING, group_body, 0)

    for k in range(RING):
        pltpu.make_async_copy(
            xbs[k], out_hbm.at[pl.ds(0, CHUNK)], sos[k]).wait()


_sc_call = functools.partial(
    pl.kernel,
    out_type=jax.ShapeDtypeStruct((N * D,), jnp.float32),
    mesh=plsc.VectorSubcoreMesh(
        core_axis_name="c", subcore_axis_name="s",
        num_cores=NC, num_subcores=NS),
    scratch_types=(
        [pltpu.VMEM((CHUNK,), jnp.float32)] * (2 * RING)
        + [pltpu.SemaphoreType.DMA] * (3 * RING)
    ),
)(_sc_body)


def _tc_add_kernel(x_ref, pos_ref, out_ref):
    out_ref[0] = x_ref[0] + pos_ref[...]


def _tc_call(x, pos_emb, nb):
    b, n, d = x.shape
    grid = (n // TC_BLK, nb)
    return pl.pallas_call(
        _tc_add_kernel,
        grid=grid,
        in_specs=[
            pl.BlockSpec((1, TC_BLK, d), lambda s, bb: (bb, s, 0)),
            pl.BlockSpec((TC_BLK, d), lambda s, bb: (s, 0)),
        ],
        out_specs=pl.BlockSpec((1, TC_BLK, d), lambda s, bb: (bb, s, 0)),
        out_shape=jax.ShapeDtypeStruct((nb, n, d), x.dtype),
    )(x, pos_emb[:n])


def kernel(x, pos_emb):
    b, n, d = x.shape
    sc_flat = _sc_call(x.reshape(-1), pos_emb[:n].reshape(-1))
    tc_out = _tc_call(x, pos_emb, SC_BATCH)
    sc_flat, tc_out = lax.optimization_barrier((sc_flat, tc_out))
    return jnp.concatenate([tc_out, sc_flat.reshape(1, n, d)], axis=0)


# speedup vs baseline: 4.6701x; 3.9471x over previous
import jax
import jax.numpy as jnp
from jax.experimental import pallas as pl


BLK = 2048


def _add_kernel(x_ref, pos_ref, out_ref):
    out_ref[0] = x_ref[0] + pos_ref[...]


def kernel(x, pos_emb):
    b, n, d = x.shape
    num_s = n // BLK
    grid = (num_s, b)
    return pl.pallas_call(
        _add_kernel,
        grid=grid,
        in_specs=[
            pl.BlockSpec((1, BLK, d), lambda s, bb: (bb, s, 0)),
            pl.BlockSpec((BLK, d), lambda s, bb: (s, 0)),
        ],
        out_specs=pl.BlockSpec((1, BLK, d), lambda s, bb: (bb, s, 0)),
        out_shape=jax.ShapeDtypeStruct((b, n, d), x.dtype),
    )(x, pos_emb[:n])
